# Initial kernel scaffold; baseline (speedup 1.0000x reference)
import jax
import jax.numpy as jnp
from jax.experimental import pallas as pl


def kernel(x, table):
    raise NotImplementedError("write your pallas kernel here")


# speedup vs baseline: 12.5552x; 12.5552x over previous
import functools

import jax
import jax.numpy as jnp
from jax import lax
from jax.experimental import pallas as pl
from jax.experimental.pallas import tpu as pltpu
from jax.experimental.pallas import tpu_sc as plsc

NUM_ROWS = 1_000_000
EMBED = 2
B = 16384 * 200
LANE = 128
NW = 32
ROWS128 = B // LANE
ROWS_PER_W = ROWS128 // NW
K = 80
NCHUNK = ROWS_PER_W // K

S1_ROWS = 2048
S1_WORDS = S1_ROWS * EMBED
S1_NCHUNKS = -(-NUM_ROWS // S1_ROWS)
S1_PER_W = -(-S1_NCHUNKS // NW)

_MESH = plsc.VectorSubcoreMesh(core_axis_name="c", subcore_axis_name="s")


_PAD = 16


@functools.partial(
    pl.kernel,
    out_type=jax.ShapeDtypeStruct((NUM_ROWS * EMBED,), jnp.float32),
    mesh=_MESH,
    scratch_types=[
        pltpu.VMEM((_PAD + S1_WORDS + _PAD,), jnp.float32),
        pltpu.VMEM((S1_WORDS,), jnp.float32),
    ],
)
def _softmax_table(tflat, qflat, buf, obuf):
    wid = lax.axis_index("s") * 2 + lax.axis_index("c")
    even = (lax.iota(jnp.int32, 16) & 1) == 0

    def chunk_body(t, carry):
        cid = jnp.minimum(wid + NW * t, S1_NCHUNKS - 1)
        start = jnp.minimum(cid * S1_ROWS, NUM_ROWS - S1_ROWS) * EMBED
        pltpu.sync_copy(tflat.at[pl.ds(start, S1_WORDS)],
                        buf.at[pl.ds(_PAD, S1_WORDS)])

        def grp(g, c2):
            base = _PAD + g * 16
            v = buf[pl.ds(base, 16)]
            nxt = buf[pl.ds(base + 1, 16)]
            prv = buf[pl.ds(base - 1, 16)]
            sw = jnp.where(even, nxt, prv)
            obuf[pl.ds(g * 16, 16)] = 1.0 / (1.0 + jnp.exp(sw - v))
            return c2

        lax.fori_loop(0, S1_WORDS // 16, grp, 0)
        pltpu.sync_copy(obuf, qflat.at[pl.ds(start, S1_WORDS)])
        return carry

    lax.fori_loop(0, S1_PER_W, chunk_body, 0)


@functools.partial(
    pl.kernel,
    out_type=jax.ShapeDtypeStruct((ROWS128, LANE, EMBED), jnp.float32),
    mesh=_MESH,
    scratch_types=[
        pltpu.VMEM((K, LANE), jnp.int32),
        pltpu.VMEM((K, LANE, EMBED), jnp.float32),
        pltpu.SemaphoreType.DMA,
    ],
    compiler_params=pltpu.CompilerParams(use_tc_tiling_on_sc=False),
)
def _gather_rows(xr, q, out, idx_v, rows_v, sem):
    wid = lax.axis_index("s") * 2 + lax.axis_index("c")

    def chunk(c, carry):
        base = wid * ROWS_PER_W + c * K
        pltpu.sync_copy(xr.at[pl.ds(base, K)], idx_v)

        def fire(j, c2):
            pltpu.make_async_copy(q.at[idx_v.at[j]], rows_v.at[j], sem).start()
            return c2

        lax.fori_loop(0, K, fire, 0)

        def drain(j, c2):
            pltpu.make_async_copy(q.at[idx_v.at[j]], rows_v.at[j], sem).wait()
            return c2

        lax.fori_loop(0, K, drain, 0)
        pltpu.sync_copy(rows_v, out.at[pl.ds(base, K)])
        return carry

    lax.fori_loop(0, NCHUNK, chunk, 0)


def kernel(x, table):
    qflat = _softmax_table(table.reshape(-1))
    q = qflat.reshape(NUM_ROWS, EMBED)
    xr = x.reshape(ROWS128, LANE)
    out = _gather_rows(xr, q)
    return out.reshape(B, EMBED)
